# baseline (speedup 1.0000x reference)
import jax
import jax.numpy as jnp
from jax.experimental import pallas as pl
from jax.experimental.pallas import tpu as pltpu


def _rgcn_body(tri_ref, feat_ref, w_ref, bias_ref, out_ref, H, G, D1, D2):
    r = pl.program_id(0)
    j = pl.program_id(1)
    nb = pl.num_programs(1)
    R = w_ref.shape[0]
    Ro = (R - 1) // 2

    @pl.when(j == 0)
    def _init():
        z = jnp.zeros(H.shape, jnp.float32)
        H[...] = z
        G[...] = z
        D1[...] = z
        D2[...] = z

    @pl.when((r == 0) & (j == 0))
    def _self_loop():
        out_ref[...] = (
            jnp.dot(feat_ref[...], w_ref[2 * Ro], preferred_element_type=jnp.float32)
            + bias_ref[...]
        )

    def body(i, carry):
        rl = tri_ref[i, 1]

        @pl.when(rl == r)
        def _():
            s = tri_ref[i, 0]
            o = tri_ref[i, 2]
            frow_o = feat_ref[pl.ds(o, 1), :]
            frow_s = feat_ref[pl.ds(s, 1), :]
            H[pl.ds(s, 1), :] += frow_o
            D1[pl.ds(s, 1), :] += 1.0
            G[pl.ds(o, 1), :] += frow_s
            D2[pl.ds(o, 1), :] += 1.0

        return carry

    jax.lax.fori_loop(0, tri_ref.shape[0], body, 0, unroll=False)

    @pl.when(j == nb - 1)
    def _flush():
        Hn = H[...] / jnp.maximum(D1[...], 1.0)
        Gn = G[...] / jnp.maximum(D2[...], 1.0)
        out_ref[...] += jnp.dot(Hn, w_ref[r], preferred_element_type=jnp.float32)
        out_ref[...] += jnp.dot(Gn, w_ref[r + Ro], preferred_element_type=jnp.float32)


def kernel(triples, features, weights, bias):
    N, F = features.shape
    R = weights.shape[0]
    Ro = (R - 1) // 2
    T = triples.shape[0]
    EB = 1000 if T % 1000 == 0 else T
    NB = T // EB
    bias2 = bias.reshape(1, F).astype(jnp.float32)

    out = pl.pallas_call(
        _rgcn_body,
        grid=(Ro, NB),
        in_specs=[
            pl.BlockSpec((EB, 3), lambda r, j: (j, 0), memory_space=pltpu.SMEM),
            pl.BlockSpec((N, F), lambda r, j: (0, 0)),
            pl.BlockSpec((R, F, F), lambda r, j: (0, 0, 0)),
            pl.BlockSpec((1, F), lambda r, j: (0, 0)),
        ],
        out_specs=pl.BlockSpec((N, F), lambda r, j: (0, 0)),
        out_shape=jax.ShapeDtypeStruct((N, F), jnp.float32),
        scratch_shapes=[pltpu.VMEM((N, F), jnp.float32)] * 4,
    )(triples, features, weights, bias2)
    return out


# speedup vs baseline: 1.4841x; 1.4841x over previous
import jax
import jax.numpy as jnp
from jax.experimental import pallas as pl
from jax.experimental.pallas import tpu as pltpu


def _rgcn_body(tri_ref, feat_ref, w_ref, bias_ref, out_ref,
               H1, G1, D11, D21, H2, G2, D12, D22):
    g = pl.program_id(0)
    j = pl.program_id(1)
    nb = pl.num_programs(1)
    R = w_ref.shape[0]
    Ro = (R - 1) // 2
    r1 = 2 * g
    r2 = 2 * g + 1

    @pl.when(j == 0)
    def _init():
        z = jnp.zeros(H1.shape, jnp.float32)
        H1[...] = z
        G1[...] = z
        D11[...] = z
        D21[...] = z
        H2[...] = z
        G2[...] = z
        D12[...] = z
        D22[...] = z

    @pl.when((g == 0) & (j == 0))
    def _self_loop():
        out_ref[...] = (
            jnp.dot(feat_ref[...], w_ref[2 * Ro], preferred_element_type=jnp.float32)
            + bias_ref[...]
        )

    def body(i, carry):
        rl = tri_ref[i, 1]

        @pl.when(rl == r1)
        def _():
            s = tri_ref[i, 0]
            o = tri_ref[i, 2]
            H1[pl.ds(s, 1), :] += feat_ref[pl.ds(o, 1), :]
            D11[pl.ds(s, 1), :] += 1.0
            G1[pl.ds(o, 1), :] += feat_ref[pl.ds(s, 1), :]
            D21[pl.ds(o, 1), :] += 1.0

        @pl.when(rl == r2)
        def _():
            s = tri_ref[i, 0]
            o = tri_ref[i, 2]
            H2[pl.ds(s, 1), :] += feat_ref[pl.ds(o, 1), :]
            D12[pl.ds(s, 1), :] += 1.0
            G2[pl.ds(o, 1), :] += feat_ref[pl.ds(s, 1), :]
            D22[pl.ds(o, 1), :] += 1.0

        return carry

    jax.lax.fori_loop(0, tri_ref.shape[0], body, 0, unroll=8)

    @pl.when(j == nb - 1)
    def _flush1():
        Hn = H1[...] / jnp.maximum(D11[...], 1.0)
        Gn = G1[...] / jnp.maximum(D21[...], 1.0)
        out_ref[...] += jnp.dot(Hn, w_ref[r1], preferred_element_type=jnp.float32)
        out_ref[...] += jnp.dot(Gn, w_ref[r1 + Ro], preferred_element_type=jnp.float32)

    @pl.when((j == nb - 1) & (r2 < Ro))
    def _flush2():
        Hn = H2[...] / jnp.maximum(D12[...], 1.0)
        Gn = G2[...] / jnp.maximum(D22[...], 1.0)
        out_ref[...] += jnp.dot(Hn, w_ref[r2], preferred_element_type=jnp.float32)
        out_ref[...] += jnp.dot(Gn, w_ref[r2 + Ro], preferred_element_type=jnp.float32)


def kernel(triples, features, weights, bias):
    N, F = features.shape
    R = weights.shape[0]
    Ro = (R - 1) // 2
    T = triples.shape[0]
    EB = 1000 if T % 1000 == 0 else T
    NB = T // EB
    bias2 = bias.reshape(1, F).astype(jnp.float32)

    out = pl.pallas_call(
        _rgcn_body,
        grid=((Ro + 1) // 2, NB),
        in_specs=[
            pl.BlockSpec((EB, 3), lambda g, j: (j, 0), memory_space=pltpu.SMEM),
            pl.BlockSpec((N, F), lambda g, j: (0, 0)),
            pl.BlockSpec((R, F, F), lambda g, j: (0, 0, 0)),
            pl.BlockSpec((1, F), lambda g, j: (0, 0)),
        ],
        out_specs=pl.BlockSpec((N, F), lambda g, j: (0, 0)),
        out_shape=jax.ShapeDtypeStruct((N, F), jnp.float32),
        scratch_shapes=[pltpu.VMEM((N, F), jnp.float32)] * 8,
    )(triples, features, weights, bias2)
    return out
